# baseline (speedup 1.0000x reference)
import functools

import jax
import jax.numpy as jnp
from jax import lax
from jax.experimental import pallas as pl
from jax.experimental.pallas import tpu as pltpu
from jax.experimental.pallas import tpu_sc as plsc

N = 10
L = 16

_mesh = plsc.VectorSubcoreMesh(core_axis_name="c", subcore_axis_name="s")


@functools.partial(
    pl.kernel,
    out_type=jax.ShapeDtypeStruct((L,), jnp.float32),
    mesh=_mesh,
    scratch_types=[
        pltpu.VMEM((L,), jnp.float32),
        pltpu.VMEM((L,), jnp.float32),
        pltpu.VMEM((L,), jnp.int32),
        pltpu.VMEM((L,), jnp.float32),
    ],
)
def _alpha_loss_sc(lb_hbm, ub_hbm, tgt_hbm, out_hbm, lb_v, ub_v, tgt_v, res_v):
    wid = lax.axis_index("s") * 2 + lax.axis_index("c")

    @pl.when(wid == 0)
    def _():
        pltpu.sync_copy(lb_hbm, lb_v)
        pltpu.sync_copy(ub_hbm, ub_v)
        pltpu.sync_copy(tgt_hbm, tgt_v)

        lb = lb_v[...]
        ub = ub_v[...]
        tgt = tgt_v[...]
        lane = lax.iota(jnp.int32, L)

        def allsum(x):
            for k in (1, 2, 4, 8):
                x = x + x.at[lane ^ k].get(mode="promise_in_bounds")
            return x

        lb_t = allsum(jnp.where(lane == tgt, lb, 0.0))
        g = jnp.where(lane == tgt, 0.0, lb_t - ub)
        g = jnp.where(lane < N, g, 0.0)
        res_v[...] = -allsum(g)
        pltpu.sync_copy(res_v, out_hbm)


def kernel(lower_bounds, upper_bounds, target):
    lb16 = jnp.pad(jnp.reshape(lower_bounds, (N,)), (0, L - N))
    ub16 = jnp.pad(jnp.reshape(upper_bounds, (N,)), (0, L - N))
    tgt16 = jnp.full((L,), target, dtype=jnp.int32)
    out = _alpha_loss_sc(lb16, ub16, tgt16)
    return out[0]


# speedup vs baseline: 1.0828x; 1.0828x over previous
import functools

import jax
import jax.numpy as jnp
from jax import lax
from jax.experimental import pallas as pl
from jax.experimental.pallas import tpu as pltpu
from jax.experimental.pallas import tpu_sc as plsc

N = 10
L = 16

_mesh = plsc.VectorSubcoreMesh(core_axis_name="c", subcore_axis_name="s")


@functools.partial(
    pl.kernel,
    out_type=jax.ShapeDtypeStruct((1,), jnp.float32),
    mesh=_mesh,
    scratch_types=[
        pltpu.VMEM((L,), jnp.float32),
        pltpu.VMEM((L,), jnp.float32),
        pltpu.VMEM((L,), jnp.int32),
        pltpu.VMEM((L,), jnp.float32),
        pltpu.SemaphoreType.DMA,
        pltpu.SemaphoreType.DMA,
        pltpu.SemaphoreType.DMA,
    ],
)
def _alpha_loss_sc(lb_hbm, ub_hbm, tgt_hbm, out_hbm, lb_v, ub_v, tgt_v, res_v,
                   sem0, sem1, sem2):
    wid = lax.axis_index("s") * 2 + lax.axis_index("c")

    @pl.when(wid == 0)
    def _():
        c0 = pltpu.async_copy(lb_hbm.at[0], lb_v.at[pl.ds(0, N)], sem0)
        c1 = pltpu.async_copy(ub_hbm.at[0], ub_v.at[pl.ds(0, N)], sem1)
        c2 = pltpu.async_copy(tgt_hbm, tgt_v.at[pl.ds(0, 1)], sem2)
        c0.wait()
        c1.wait()
        c2.wait()

        lb = lb_v[...]
        ub = ub_v[...]
        tgt = tgt_v[...][0]
        lane = lax.iota(jnp.int32, L)

        def allsum(x):
            for k in (1, 2, 4, 8):
                x = x + x.at[lane ^ k].get(mode="promise_in_bounds")
            return x

        lb_t = allsum(jnp.where(lane == tgt, lb, 0.0))
        g = jnp.where(lane == tgt, 0.0, lb_t - ub)
        g = jnp.where(lane < N, g, 0.0)
        res_v[...] = -allsum(g)
        pltpu.sync_copy(res_v.at[pl.ds(0, 1)], out_hbm)


def kernel(lower_bounds, upper_bounds, target):
    tgt1 = jnp.reshape(jnp.asarray(target, dtype=jnp.int32), (1,))
    out = _alpha_loss_sc(lower_bounds, upper_bounds, tgt1)
    return jnp.reshape(out, ())


# speedup vs baseline: 1.1797x; 1.0895x over previous
import functools

import jax
import jax.numpy as jnp
from jax import lax
from jax.experimental import pallas as pl
from jax.experimental.pallas import tpu as pltpu
from jax.experimental.pallas import tpu_sc as plsc

N = 10
L = 16

_mesh = plsc.VectorSubcoreMesh(
    core_axis_name="c", subcore_axis_name="s", num_cores=1, num_subcores=1
)


@functools.partial(
    pl.kernel,
    out_type=jax.ShapeDtypeStruct((1,), jnp.float32),
    mesh=_mesh,
    scratch_types=[
        pltpu.VMEM((L,), jnp.float32),
        pltpu.VMEM((L,), jnp.float32),
        pltpu.VMEM((L,), jnp.int32),
        pltpu.VMEM((L,), jnp.float32),
        pltpu.SemaphoreType.DMA,
        pltpu.SemaphoreType.DMA,
        pltpu.SemaphoreType.DMA,
    ],
)
def _alpha_loss_sc(lb_hbm, ub_hbm, tgt_hbm, out_hbm, lb_v, ub_v, tgt_v, res_v,
                   sem0, sem1, sem2):
    wid = lax.axis_index("s") * 2 + lax.axis_index("c")

    @pl.when(wid == 0)
    def _():
        c0 = pltpu.async_copy(lb_hbm.at[0], lb_v.at[pl.ds(0, N)], sem0)
        c1 = pltpu.async_copy(ub_hbm.at[0], ub_v.at[pl.ds(0, N)], sem1)
        c2 = pltpu.async_copy(tgt_hbm, tgt_v.at[pl.ds(0, 1)], sem2)
        c0.wait()
        c1.wait()
        c2.wait()

        lb = lb_v[...]
        ub = ub_v[...]
        tgt = tgt_v[...][0]
        lane = lax.iota(jnp.int32, L)

        def allsum(x):
            for k in (1, 2, 4, 8):
                x = x + x.at[lane ^ k].get(mode="promise_in_bounds")
            return x

        lb_t = allsum(jnp.where(lane == tgt, lb, 0.0))
        g = jnp.where(lane == tgt, 0.0, lb_t - ub)
        g = jnp.where(lane < N, g, 0.0)
        res_v[...] = -allsum(g)
        pltpu.sync_copy(res_v.at[pl.ds(0, 1)], out_hbm)


def kernel(lower_bounds, upper_bounds, target):
    tgt1 = jnp.reshape(jnp.asarray(target, dtype=jnp.int32), (1,))
    out = _alpha_loss_sc(lower_bounds, upper_bounds, tgt1)
    return jnp.reshape(out, ())
